# Initial kernel scaffold; baseline (speedup 1.0000x reference)
import functools

import jax
import jax.numpy as jnp
from jax import lax
from jax.experimental import pallas as pl
from jax.experimental.pallas import tpu as pltpu
from jax.experimental.pallas import tpu_sc as plsc

N = 16384
B = 16
LANES = 16
NB = 4
NUM_TILES = 32
BATCH_GROUPS = B // NB
NUM_RANGES = NUM_TILES // BATCH_GROUPS
CHUNK = 2048


@functools.partial(jax.jit, static_argnames=("per_range",))
def _sc_partials(z, rows_p, cols_p, vals_p, per_range):
    nchunks = per_range // CHUNK
    mesh = plsc.VectorSubcoreMesh(
        core_axis_name="c", subcore_axis_name="s", num_cores=2, num_subcores=16
    )

    @functools.partial(
        pl.kernel,
        out_type=jax.ShapeDtypeStruct((NUM_TILES, NB, LANES), jnp.float32),
        mesh=mesh,
        scratch_types=[
            pltpu.VMEM((N,), jnp.float32),
            pltpu.VMEM((N,), jnp.float32),
            pltpu.VMEM((N,), jnp.float32),
            pltpu.VMEM((N,), jnp.float32),
            pltpu.VMEM((CHUNK,), jnp.int32),
            pltpu.VMEM((CHUNK,), jnp.int32),
            pltpu.VMEM((CHUNK,), jnp.float32),
            pltpu.VMEM((NB, LANES), jnp.float32),
        ],
    )
    def body(z_hbm, rows_hbm, cols_hbm, vals_hbm, out_hbm,
             z0, z1, z2, z3, rbuf, cbuf, vbuf, accv):
        c = lax.axis_index("c")
        s = lax.axis_index("s")
        wid = c * 16 + s
        g = wid // NUM_RANGES
        rid = wid % NUM_RANGES
        zrefs = [z0, z1, z2, z3]
        for b in range(NB):
            pltpu.sync_copy(z_hbm.at[g * NB + b], zrefs[b])
        base = rid * per_range

        def chunk_body(k, accs):
            off = base + k * CHUNK
            pltpu.sync_copy(rows_hbm.at[pl.ds(off, CHUNK)], rbuf)
            pltpu.sync_copy(cols_hbm.at[pl.ds(off, CHUNK)], cbuf)
            pltpu.sync_copy(vals_hbm.at[pl.ds(off, CHUNK)], vbuf)

            def grp_body(j, accs):
                ir = rbuf[pl.ds(j * LANES, LANES)]
                ic = cbuf[pl.ds(j * LANES, LANES)]
                v = vbuf[pl.ds(j * LANES, LANES)]
                out = []
                for b in range(NB):
                    zr = plsc.load_gather(zrefs[b], [ir])
                    zc = plsc.load_gather(zrefs[b], [ic])
                    out.append(accs[b] + v * zr * zc)
                return tuple(out)

            return lax.fori_loop(0, CHUNK // LANES, grp_body, accs)

        zero = jnp.zeros((LANES,), jnp.float32)
        accs = lax.fori_loop(0, nchunks, chunk_body, (zero,) * NB)
        for b in range(NB):
            accv[b, :] = accs[b]
        pltpu.sync_copy(accv, out_hbm.at[wid])

    return body(z, rows_p, cols_p, vals_p)


def kernel(z, rows, cols, vals):
    nnz = rows.shape[0]
    per_range = -(-nnz // (NUM_RANGES * CHUNK)) * CHUNK
    pad = per_range * NUM_RANGES - nnz
    rows_p = jnp.concatenate([rows.astype(jnp.int32),
                              jnp.zeros((pad,), jnp.int32)])
    cols_p = jnp.concatenate([cols.astype(jnp.int32),
                              jnp.zeros((pad,), jnp.int32)])
    vals_p = jnp.concatenate([vals, jnp.zeros((pad,), jnp.float32)])
    partials = _sc_partials(z, rows_p, cols_p, vals_p, per_range)
    per_tile = partials.sum(axis=2)
    totals = per_tile.reshape(BATCH_GROUPS, NUM_RANGES, NB).sum(axis=1)
    return jnp.mean(jnp.sqrt(totals.reshape(B)))


# speedup vs baseline: 38.3020x; 38.3020x over previous
import functools

import jax
import jax.numpy as jnp
from jax import lax
from jax.experimental import pallas as pl
from jax.experimental.pallas import tpu as pltpu
from jax.experimental.pallas import tpu_sc as plsc

N = 16384
B = 16
LANES = 16
NB = 4
NUM_TILES = 32
BATCH_GROUPS = B // NB
NUM_RANGES = NUM_TILES // BATCH_GROUPS
CHUNK = 2048


@functools.partial(jax.jit, static_argnames=("per_range",))
def _sc_partials(z, rows_p, cols_p, vals_p, per_range):
    nchunks = per_range // CHUNK
    mesh = plsc.VectorSubcoreMesh(
        core_axis_name="c", subcore_axis_name="s", num_cores=2, num_subcores=16
    )

    @functools.partial(
        pl.kernel,
        out_type=jax.ShapeDtypeStruct((NUM_TILES, NB, LANES), jnp.float32),
        mesh=mesh,
        compiler_params=pltpu.CompilerParams(needs_layout_passes=False),
        scratch_types=[
            pltpu.VMEM((N,), jnp.float32),
            pltpu.VMEM((N,), jnp.float32),
            pltpu.VMEM((N,), jnp.float32),
            pltpu.VMEM((N,), jnp.float32),
            pltpu.VMEM((CHUNK,), jnp.int32),
            pltpu.VMEM((CHUNK,), jnp.int32),
            pltpu.VMEM((CHUNK,), jnp.float32),
            pltpu.VMEM((NB, LANES), jnp.float32),
        ],
    )
    def body(z_hbm, rows_hbm, cols_hbm, vals_hbm, out_hbm,
             z0, z1, z2, z3, rbuf, cbuf, vbuf, accv):
        c = lax.axis_index("c")
        s = lax.axis_index("s")
        wid = c * 16 + s
        g = wid // NUM_RANGES
        rid = wid % NUM_RANGES
        zrefs = [z0, z1, z2, z3]
        for b in range(NB):
            pltpu.sync_copy(z_hbm.at[g * NB + b], zrefs[b])
        base = rid * per_range

        def chunk_body(k, accs):
            off = base + k * CHUNK
            pltpu.sync_copy(rows_hbm.at[pl.ds(off, CHUNK)], rbuf)
            pltpu.sync_copy(cols_hbm.at[pl.ds(off, CHUNK)], cbuf)
            pltpu.sync_copy(vals_hbm.at[pl.ds(off, CHUNK)], vbuf)

            def grp_body(j, accs):
                ir = rbuf[pl.ds(j * LANES, LANES)]
                ic = cbuf[pl.ds(j * LANES, LANES)]
                v = vbuf[pl.ds(j * LANES, LANES)]
                out = []
                for b in range(NB):
                    zr = plsc.load_gather(zrefs[b], [ir])
                    zc = plsc.load_gather(zrefs[b], [ic])
                    out.append(accs[b] + v * zr * zc)
                return tuple(out)

            return lax.fori_loop(0, CHUNK // LANES, grp_body, accs)

        zero = jnp.zeros((LANES,), jnp.float32)
        accs = lax.fori_loop(0, nchunks, chunk_body, (zero,) * NB)
        for b in range(NB):
            accv[b, :] = accs[b]
        pltpu.sync_copy(accv, out_hbm.at[wid])

    return body(z, rows_p, cols_p, vals_p)


def kernel(z, rows, cols, vals):
    nnz = rows.shape[0]
    per_range = -(-nnz // (NUM_RANGES * CHUNK)) * CHUNK
    pad = per_range * NUM_RANGES - nnz
    rows_p = jnp.concatenate([rows.astype(jnp.int32),
                              jnp.zeros((pad,), jnp.int32)])
    cols_p = jnp.concatenate([cols.astype(jnp.int32),
                              jnp.zeros((pad,), jnp.int32)])
    vals_p = jnp.concatenate([vals, jnp.zeros((pad,), jnp.float32)])
    partials = _sc_partials(z, rows_p, cols_p, vals_p, per_range)
    per_tile = partials.sum(axis=2)
    totals = per_tile.reshape(BATCH_GROUPS, NUM_RANGES, NB).sum(axis=1)
    return jnp.mean(jnp.sqrt(totals.reshape(B)))


# speedup vs baseline: 88.0827x; 2.2997x over previous
import functools

import jax
import jax.numpy as jnp
from jax import lax
from jax.experimental import pallas as pl
from jax.experimental.pallas import tpu as pltpu
from jax.experimental.pallas import tpu_sc as plsc

N = 16384
B = 16
LANES = 16
NB = 4
NUM_TILES = 32
BATCH_GROUPS = B // NB
NUM_RANGES = NUM_TILES // BATCH_GROUPS
CHUNK = 4096
UNROLL = 8


@functools.partial(jax.jit, static_argnames=("per_range",))
def _sc_partials(z, rc_p, vals_p, per_range):
    nc2 = per_range // (2 * CHUNK)
    gpc = CHUNK // LANES
    mesh = plsc.VectorSubcoreMesh(
        core_axis_name="c", subcore_axis_name="s", num_cores=2, num_subcores=16
    )

    @functools.partial(
        pl.kernel,
        out_type=jax.ShapeDtypeStruct((NUM_TILES, NB, LANES), jnp.float32),
        mesh=mesh,
        compiler_params=pltpu.CompilerParams(needs_layout_passes=False),
        scratch_types=[
            pltpu.VMEM((N,), jnp.float32),
            pltpu.VMEM((N,), jnp.float32),
            pltpu.VMEM((N,), jnp.float32),
            pltpu.VMEM((N,), jnp.float32),
            pltpu.VMEM((CHUNK,), jnp.int32),
            pltpu.VMEM((CHUNK,), jnp.int32),
            pltpu.VMEM((CHUNK,), jnp.float32),
            pltpu.VMEM((CHUNK,), jnp.float32),
            pltpu.VMEM((NB, LANES), jnp.float32),
            pltpu.SemaphoreType.DMA,
            pltpu.SemaphoreType.DMA,
            pltpu.SemaphoreType.DMA,
            pltpu.SemaphoreType.DMA,
        ],
    )
    def body(z_hbm, rc_hbm, vals_hbm, out_hbm,
             z0, z1, z2, z3, rc0, rc1, v0, v1, accv,
             sa_rc, sa_v, sb_rc, sb_v):
        c = lax.axis_index("c")
        s = lax.axis_index("s")
        wid = c * 16 + s
        g = wid // NUM_RANGES
        rid = wid % NUM_RANGES
        zrefs = [z0, z1, z2, z3]
        for b in range(NB):
            pltpu.sync_copy(z_hbm.at[g * NB + b], zrefs[b])
        base = rid * per_range

        def start(cidx, rcb, vb, sem_rc, sem_v):
            off = base + cidx * CHUNK
            pltpu.async_copy(rc_hbm.at[pl.ds(off, CHUNK)], rcb, sem_rc)
            pltpu.async_copy(vals_hbm.at[pl.ds(off, CHUNK)], vb, sem_v)

        def wait(cidx, rcb, vb, sem_rc, sem_v):
            off = base + cidx * CHUNK
            pltpu.make_async_copy(rc_hbm.at[pl.ds(off, CHUNK)], rcb, sem_rc).wait()
            pltpu.make_async_copy(vals_hbm.at[pl.ds(off, CHUNK)], vb, sem_v).wait()

        def process(rcb, vb, accs):
            def grp(jj, accs):
                accs = list(accs)
                for u in range(UNROLL):
                    goff = (jj * UNROLL + u) * LANES
                    rc = rcb[pl.ds(goff, LANES)]
                    v = vb[pl.ds(goff, LANES)]
                    ir = lax.shift_right_logical(rc, 14)
                    ic = lax.bitwise_and(rc, 16383)
                    p = u % 2
                    for b in range(NB):
                        zr = plsc.load_gather(zrefs[b], [ir])
                        zc = plsc.load_gather(zrefs[b], [ic])
                        accs[2 * b + p] = accs[2 * b + p] + v * zr * zc
                return tuple(accs)

            return lax.fori_loop(0, gpc // UNROLL, grp, accs)

        start(0, rc0, v0, sa_rc, sa_v)

        def body2(k2, accs):
            c0 = 2 * k2
            start(c0 + 1, rc1, v1, sb_rc, sb_v)
            wait(c0, rc0, v0, sa_rc, sa_v)
            accs = process(rc0, v0, accs)

            @pl.when(k2 + 1 < nc2)
            def _():
                start(c0 + 2, rc0, v0, sa_rc, sa_v)

            wait(c0 + 1, rc1, v1, sb_rc, sb_v)
            accs = process(rc1, v1, accs)
            return accs

        zero = jnp.zeros((LANES,), jnp.float32)
        accs = lax.fori_loop(0, nc2, body2, (zero,) * (2 * NB))
        for b in range(NB):
            accv[b, :] = accs[2 * b] + accs[2 * b + 1]
        pltpu.sync_copy(accv, out_hbm.at[wid])

    return body(z, rc_p, vals_p)


def kernel(z, rows, cols, vals):
    nnz = rows.shape[0]
    per_range = -(-nnz // (NUM_RANGES * 2 * CHUNK)) * (2 * CHUNK)
    pad = per_range * NUM_RANGES - nnz
    rc = rows.astype(jnp.int32) * N + cols.astype(jnp.int32)
    rc_p = jnp.concatenate([rc, jnp.zeros((pad,), jnp.int32)])
    vals_p = jnp.concatenate([vals, jnp.zeros((pad,), jnp.float32)])
    partials = _sc_partials(z, rc_p, vals_p, per_range)
    per_tile = partials.sum(axis=2)
    totals = per_tile.reshape(BATCH_GROUPS, NUM_RANGES, NB).sum(axis=1)
    return jnp.mean(jnp.sqrt(totals.reshape(B)))


# speedup vs baseline: 156.7336x; 1.7794x over previous
import functools

import jax
import jax.numpy as jnp
from jax import lax
from jax.experimental import pallas as pl
from jax.experimental.pallas import tpu as pltpu
from jax.experimental.pallas import tpu_sc as plsc

N = 16384
B = 16
LANES = 16
NB = 8
NP = NB // 2
NUM_TILES = 32
BATCH_GROUPS = B // NB
NUM_RANGES = NUM_TILES // BATCH_GROUPS
CHUNK = 4096
UNROLL = 4


@functools.partial(jax.jit, static_argnames=("per_range",))
def _sc_partials(zz, rc_p, vals_p, per_range):
    nc2 = per_range // (2 * CHUNK)
    gpc = CHUNK // LANES
    mesh = plsc.VectorSubcoreMesh(
        core_axis_name="c", subcore_axis_name="s", num_cores=2, num_subcores=16
    )

    @functools.partial(
        pl.kernel,
        out_type=jax.ShapeDtypeStruct((NUM_TILES, NB, LANES), jnp.float32),
        mesh=mesh,
        compiler_params=pltpu.CompilerParams(needs_layout_passes=False),
        scratch_types=[
            pltpu.VMEM((N,), jnp.int32),
            pltpu.VMEM((N,), jnp.int32),
            pltpu.VMEM((N,), jnp.int32),
            pltpu.VMEM((N,), jnp.int32),
            pltpu.VMEM((CHUNK,), jnp.int32),
            pltpu.VMEM((CHUNK,), jnp.int32),
            pltpu.VMEM((CHUNK,), jnp.float32),
            pltpu.VMEM((CHUNK,), jnp.float32),
            pltpu.VMEM((NB, LANES), jnp.float32),
            pltpu.SemaphoreType.DMA,
            pltpu.SemaphoreType.DMA,
            pltpu.SemaphoreType.DMA,
            pltpu.SemaphoreType.DMA,
        ],
    )
    def body(zz_hbm, rc_hbm, vals_hbm, out_hbm,
             zz0, zz1, zz2, zz3, rc0, rc1, v0, v1, accv,
             sa_rc, sa_v, sb_rc, sb_v):
        c = lax.axis_index("c")
        s = lax.axis_index("s")
        wid = c * 16 + s
        g = wid // NUM_RANGES
        rid = wid % NUM_RANGES
        zzrefs = [zz0, zz1, zz2, zz3]
        for p in range(NP):
            pltpu.sync_copy(zz_hbm.at[g * NP + p], zzrefs[p])
        base = rid * per_range

        def start(cidx, rcb, vb, sem_rc, sem_v):
            off = base + cidx * CHUNK
            pltpu.async_copy(rc_hbm.at[pl.ds(off, CHUNK)], rcb, sem_rc)
            pltpu.async_copy(vals_hbm.at[pl.ds(off, CHUNK)], vb, sem_v)

        def wait(cidx, rcb, vb, sem_rc, sem_v):
            off = base + cidx * CHUNK
            pltpu.make_async_copy(rc_hbm.at[pl.ds(off, CHUNK)], rcb, sem_rc).wait()
            pltpu.make_async_copy(vals_hbm.at[pl.ds(off, CHUNK)], vb, sem_v).wait()

        def process(rcb, vb, accs):
            def grp(jj, accs):
                accs = list(accs)
                for u in range(UNROLL):
                    goff = (jj * UNROLL + u) * LANES
                    rcv = rcb[pl.ds(goff, LANES)]
                    v = vb[pl.ds(goff, LANES)]
                    ir = lax.shift_right_logical(rcv, 14)
                    ic = lax.bitwise_and(rcv, 16383)
                    for p in range(NP):
                        zr = plsc.load_gather(zzrefs[p], [ir])
                        zc = plsc.load_gather(zzrefs[p], [ic])
                        prod = (plsc.bitcast(zr, jnp.bfloat16)
                                * plsc.bitcast(zc, jnp.bfloat16))
                        pa, pb = plsc.unpack(prod,
                                             format=plsc.PackFormat.INTERLEAVED)
                        accs[2 * p] = accs[2 * p] + v * pa
                        accs[2 * p + 1] = accs[2 * p + 1] + v * pb
                return tuple(accs)

            return lax.fori_loop(0, gpc // UNROLL, grp, accs)

        start(0, rc0, v0, sa_rc, sa_v)

        def body2(k2, accs):
            c0 = 2 * k2
            start(c0 + 1, rc1, v1, sb_rc, sb_v)
            wait(c0, rc0, v0, sa_rc, sa_v)
            accs = process(rc0, v0, accs)

            @pl.when(k2 + 1 < nc2)
            def _():
                start(c0 + 2, rc0, v0, sa_rc, sa_v)

            wait(c0 + 1, rc1, v1, sb_rc, sb_v)
            accs = process(rc1, v1, accs)
            return accs

        zero = jnp.zeros((LANES,), jnp.float32)
        accs = lax.fori_loop(0, nc2, body2, (zero,) * NB)
        for b in range(NB):
            accv[b, :] = accs[b]
        pltpu.sync_copy(accv, out_hbm.at[wid])

    return body(zz, rc_p, vals_p)


def kernel(z, rows, cols, vals):
    nnz = rows.shape[0]
    per_range = -(-nnz // (NUM_RANGES * 2 * CHUNK)) * (2 * CHUNK)
    pad = per_range * NUM_RANGES - nnz
    rc = rows.astype(jnp.int32) * N + cols.astype(jnp.int32)
    rc_p = jnp.concatenate([rc, jnp.zeros((pad,), jnp.int32)])
    vals_p = jnp.concatenate([vals, jnp.zeros((pad,), jnp.float32)])
    zb = jax.lax.bitcast_convert_type(z.astype(jnp.bfloat16), jnp.uint16)
    lo = zb[0::2].astype(jnp.uint32)
    hi = zb[1::2].astype(jnp.uint32)
    zz = jax.lax.bitcast_convert_type(lo | (hi << 16), jnp.int32)
    partials = _sc_partials(zz, rc_p, vals_p, per_range)
    per_tile = partials.sum(axis=2)
    totals = per_tile.reshape(BATCH_GROUPS, NUM_RANGES, NB).sum(axis=1)
    return jnp.mean(jnp.sqrt(totals.reshape(B)))


# speedup vs baseline: 161.9371x; 1.0332x over previous
import functools

import jax
import jax.numpy as jnp
from jax import lax
from jax.experimental import pallas as pl
from jax.experimental.pallas import tpu as pltpu
from jax.experimental.pallas import tpu_sc as plsc

N = 16384
B = 16
LANES = 16
NB = 8
NP = NB // 2
NUM_TILES = 32
BATCH_GROUPS = B // NB
NUM_RANGES = NUM_TILES // BATCH_GROUPS
CHUNK = 4096
UNROLL = 4


@functools.partial(jax.jit, static_argnames=("per_range", "nnz"))
def _sc_partials(zz, rc, vals, rc_tail, vals_tail, per_range, nnz):
    nc2 = per_range // (2 * CHUNK)
    last_base = (NUM_RANGES - 1) * per_range
    last_len = nnz - last_base
    last_full = last_len // CHUNK
    last_nc2 = last_full // 2
    last_odd = last_full % 2
    gpc = CHUNK // LANES
    mesh = plsc.VectorSubcoreMesh(
        core_axis_name="c", subcore_axis_name="s", num_cores=2, num_subcores=16
    )

    @functools.partial(
        pl.kernel,
        out_type=jax.ShapeDtypeStruct((NUM_TILES, NB, LANES), jnp.float32),
        mesh=mesh,
        compiler_params=pltpu.CompilerParams(needs_layout_passes=False),
        scratch_types=[
            pltpu.VMEM((N,), jnp.int32),
            pltpu.VMEM((N,), jnp.int32),
            pltpu.VMEM((N,), jnp.int32),
            pltpu.VMEM((N,), jnp.int32),
            pltpu.VMEM((CHUNK,), jnp.int32),
            pltpu.VMEM((CHUNK,), jnp.int32),
            pltpu.VMEM((CHUNK,), jnp.float32),
            pltpu.VMEM((CHUNK,), jnp.float32),
            pltpu.VMEM((NB, LANES), jnp.float32),
            pltpu.SemaphoreType.DMA,
            pltpu.SemaphoreType.DMA,
            pltpu.SemaphoreType.DMA,
            pltpu.SemaphoreType.DMA,
        ],
    )
    def body(zz_hbm, rc_hbm, vals_hbm, rct_hbm, vt_hbm, out_hbm,
             zz0, zz1, zz2, zz3, rc0, rc1, v0, v1, accv,
             sa_rc, sa_v, sb_rc, sb_v):
        c = lax.axis_index("c")
        s = lax.axis_index("s")
        wid = c * 16 + s
        g = wid // NUM_RANGES
        rid = wid % NUM_RANGES
        is_last = rid == NUM_RANGES - 1
        zzrefs = [zz0, zz1, zz2, zz3]
        for p in range(NP):
            pltpu.sync_copy(zz_hbm.at[g * NP + p], zzrefs[p])
        base = rid * per_range

        def start(cidx, rcb, vb, sem_rc, sem_v):
            off = base + cidx * CHUNK
            pltpu.async_copy(rc_hbm.at[pl.ds(off, CHUNK)], rcb, sem_rc)
            pltpu.async_copy(vals_hbm.at[pl.ds(off, CHUNK)], vb, sem_v)

        def wait(cidx, rcb, vb, sem_rc, sem_v):
            off = base + cidx * CHUNK
            pltpu.make_async_copy(rc_hbm.at[pl.ds(off, CHUNK)], rcb, sem_rc).wait()
            pltpu.make_async_copy(vals_hbm.at[pl.ds(off, CHUNK)], vb, sem_v).wait()

        def process(rcb, vb, accs):
            def grp(jj, accs):
                accs = list(accs)
                for u in range(UNROLL):
                    goff = (jj * UNROLL + u) * LANES
                    rcv = rcb[pl.ds(goff, LANES)]
                    v = vb[pl.ds(goff, LANES)]
                    ir = lax.shift_right_logical(rcv, 14)
                    ic = lax.bitwise_and(rcv, 16383)
                    for p in range(NP):
                        zr = plsc.load_gather(zzrefs[p], [ir])
                        zc = plsc.load_gather(zzrefs[p], [ic])
                        prod = (plsc.bitcast(zr, jnp.bfloat16)
                                * plsc.bitcast(zc, jnp.bfloat16))
                        pa, pb = plsc.unpack(prod,
                                             format=plsc.PackFormat.INTERLEAVED)
                        accs[2 * p] = accs[2 * p] + v * pa
                        accs[2 * p + 1] = accs[2 * p + 1] + v * pb
                return tuple(accs)

            return lax.fori_loop(0, gpc // UNROLL, grp, accs)

        my_nc2 = jnp.where(is_last, last_nc2, nc2)
        start(0, rc0, v0, sa_rc, sa_v)

        def body2(k2, accs):
            c0 = 2 * k2
            start(c0 + 1, rc1, v1, sb_rc, sb_v)
            wait(c0, rc0, v0, sa_rc, sa_v)
            accs = process(rc0, v0, accs)

            @pl.when(k2 + 1 < my_nc2)
            def _():
                start(c0 + 2, rc0, v0, sa_rc, sa_v)

            wait(c0 + 1, rc1, v1, sb_rc, sb_v)
            accs = process(rc1, v1, accs)
            return accs

        zero = jnp.zeros((LANES,), jnp.float32)
        accs = lax.fori_loop(0, my_nc2, body2, (zero,) * NB)

        def tail_work(accs_l):
            if last_odd:
                offc = base + (last_full - 1) * CHUNK
                pltpu.sync_copy(rc_hbm.at[pl.ds(offc, CHUNK)], rc0)
                pltpu.sync_copy(vals_hbm.at[pl.ds(offc, CHUNK)], v0)
                accs_l = process(rc0, v0, accs_l)
            pltpu.sync_copy(rct_hbm, rc1)
            pltpu.sync_copy(vt_hbm, v1)
            return process(rc1, v1, accs_l)

        accs = lax.cond(is_last, tail_work, lambda a: tuple(a), accs)

        for b in range(NB):
            accv[b, :] = accs[b]
        pltpu.sync_copy(accv, out_hbm.at[wid])

    return body(zz, rc, vals, rc_tail, vals_tail)


def kernel(z, rows, cols, vals):
    nnz = rows.shape[0]
    per_range = -(-(nnz // NUM_RANGES) // (2 * CHUNK)) * (2 * CHUNK)
    rc = rows.astype(jnp.int32) * N + cols.astype(jnp.int32)
    last_base = (NUM_RANGES - 1) * per_range
    cut = last_base + ((nnz - last_base) // CHUNK) * CHUNK
    tail_n = nnz - cut
    rc_tail = jnp.zeros((CHUNK,), jnp.int32).at[:tail_n].set(rc[cut:])
    vals_tail = jnp.zeros((CHUNK,), jnp.float32).at[:tail_n].set(vals[cut:])
    zb = jax.lax.bitcast_convert_type(z.astype(jnp.bfloat16), jnp.uint16)
    lo = zb[0::2].astype(jnp.uint32)
    hi = zb[1::2].astype(jnp.uint32)
    zz = jax.lax.bitcast_convert_type(lo | (hi << 16), jnp.int32)
    partials = _sc_partials(zz, rc, vals, rc_tail, vals_tail, per_range, nnz)
    per_tile = partials.sum(axis=2)
    totals = per_tile.reshape(BATCH_GROUPS, NUM_RANGES, NB).sum(axis=1)
    return jnp.mean(jnp.sqrt(totals.reshape(B)))
